# Initial kernel scaffold; baseline (speedup 1.0000x reference)
import jax
import jax.numpy as jnp
from jax.experimental import pallas as pl


def kernel(x, A, D, concepts_q, concepts_c):
    raise NotImplementedError("write your pallas kernel here")


# speedup vs baseline: 2.8721x; 2.8721x over previous
import functools

import jax
import jax.numpy as jnp
from jax.experimental import pallas as pl

_NUM_STUDENTS = 4096
_NUM_QUESTIONS = 16384
_NUM_CONCEPTS = 128
_QB = 512


def _fwd(a_ref, d_ref, c_ref, o_ref):
    c = c_ref[0]
    oh = (c == jax.lax.broadcasted_iota(jnp.int32, (_NUM_CONCEPTS, _QB), 0))
    oh = oh.astype(jnp.bfloat16)
    a = jnp.dot(a_ref[...].astype(jnp.bfloat16), oh,
                preferred_element_type=jnp.float32)
    m = jnp.dot(d_ref[...].astype(jnp.bfloat16), oh,
                preferred_element_type=jnp.float32)
    qi = jax.lax.broadcasted_iota(jnp.int32, (_QB, _QB), 0)
    qj = jax.lax.broadcasted_iota(jnp.int32, (_QB, _QB), 1)
    d_row = jnp.sum(jnp.where(qi == qj, m, 0.0), axis=0, keepdims=True)
    z = 0.5 * (a - d_row)
    o_ref[...] = 0.6 + 0.4 * jnp.tanh(z)


def kernel(x, A, D, concepts_q, concepts_c):
    nb = _NUM_QUESTIONS // _QB
    c3 = concepts_c.reshape(nb, 1, _QB)
    return pl.pallas_call(
        _fwd,
        grid=(nb,),
        in_specs=[
            pl.BlockSpec((_NUM_STUDENTS, _NUM_CONCEPTS), lambda q: (0, 0)),
            pl.BlockSpec((_QB, _NUM_CONCEPTS), lambda q: (q, 0)),
            pl.BlockSpec((1, 1, _QB), lambda q: (q, 0, 0)),
        ],
        out_specs=pl.BlockSpec((_NUM_STUDENTS, _QB), lambda q: (0, q)),
        out_shape=jax.ShapeDtypeStruct((_NUM_STUDENTS, _NUM_QUESTIONS),
                                       jnp.float32),
    )(A, D, c3)


# speedup vs baseline: 4.9711x; 1.7308x over previous
import jax
import jax.numpy as jnp
from jax.experimental import pallas as pl

_NUM_STUDENTS = 4096
_NUM_QUESTIONS = 16384
_NUM_CONCEPTS = 128
_QB = 512


def _fwd(a_ref, d_ref, c_ref, o_ref):
    c = c_ref[0]
    oh = (c == jax.lax.broadcasted_iota(jnp.int32, (_NUM_CONCEPTS, _QB), 0))
    oh = oh.astype(jnp.bfloat16)
    m = jnp.dot(d_ref[...], oh, preferred_element_type=jnp.float32)
    qi = jax.lax.broadcasted_iota(jnp.int32, (_QB, _QB), 0)
    qj = jax.lax.broadcasted_iota(jnp.int32, (_QB, _QB), 1)
    neg_dh = -jnp.sum(jnp.where(qi == qj, m, 0.0), axis=0, keepdims=True)
    neg_dh = neg_dh.astype(jnp.bfloat16)
    pad = jnp.zeros((_NUM_CONCEPTS - 1, _QB), jnp.bfloat16)
    oh_aug = jnp.concatenate([oh, neg_dh, pad], axis=0)
    z = jnp.dot(a_ref[...], oh_aug, preferred_element_type=jnp.float32)
    o_ref[...] = 0.6 + 0.4 * jnp.tanh(z)


def kernel(x, A, D, concepts_q, concepts_c):
    nb = _NUM_QUESTIONS // _QB
    c3 = concepts_c.reshape(nb, 1, _QB)
    a_aug = jnp.zeros((_NUM_STUDENTS, 2 * _NUM_CONCEPTS), jnp.bfloat16)
    a_aug = a_aug.at[:, :_NUM_CONCEPTS].set((0.5 * A).astype(jnp.bfloat16))
    a_aug = a_aug.at[:, _NUM_CONCEPTS].set(jnp.bfloat16(1.0))
    d_half = (0.5 * D).astype(jnp.bfloat16)
    return pl.pallas_call(
        _fwd,
        grid=(nb,),
        in_specs=[
            pl.BlockSpec((_NUM_STUDENTS, 2 * _NUM_CONCEPTS), lambda q: (0, 0)),
            pl.BlockSpec((_QB, _NUM_CONCEPTS), lambda q: (q, 0)),
            pl.BlockSpec((1, 1, _QB), lambda q: (q, 0, 0)),
        ],
        out_specs=pl.BlockSpec((_NUM_STUDENTS, _QB), lambda q: (0, q)),
        out_shape=jax.ShapeDtypeStruct((_NUM_STUDENTS, _NUM_QUESTIONS),
                                       jnp.float32),
    )(a_aug, d_half, c3)


# speedup vs baseline: 4.9977x; 1.0054x over previous
import jax
import jax.numpy as jnp
from jax.experimental import pallas as pl

_NUM_STUDENTS = 4096
_NUM_QUESTIONS = 16384
_NUM_CONCEPTS = 128
_QB = 1024


def _fwd(a_ref, d_ref, c_ref, o_ref):
    c = c_ref[0]
    oh = (c == jax.lax.broadcasted_iota(jnp.int32, (_NUM_CONCEPTS, _QB), 0))
    oh = oh.astype(jnp.bfloat16)
    m = jnp.dot(d_ref[...], oh, preferred_element_type=jnp.float32)
    qi = jax.lax.broadcasted_iota(jnp.int32, (_QB, _QB), 0)
    qj = jax.lax.broadcasted_iota(jnp.int32, (_QB, _QB), 1)
    neg_dh = -jnp.sum(jnp.where(qi == qj, m, 0.0), axis=0, keepdims=True)
    neg_dh = neg_dh.astype(jnp.bfloat16)
    pad = jnp.zeros((_NUM_CONCEPTS - 1, _QB), jnp.bfloat16)
    oh_aug = jnp.concatenate([oh, neg_dh, pad], axis=0)
    z = jnp.dot(a_ref[...], oh_aug, preferred_element_type=jnp.float32)
    o_ref[...] = 0.6 + 0.4 * jnp.tanh(z)


def kernel(x, A, D, concepts_q, concepts_c):
    nb = _NUM_QUESTIONS // _QB
    c3 = concepts_c.reshape(nb, 1, _QB)
    a_aug = jnp.zeros((_NUM_STUDENTS, 2 * _NUM_CONCEPTS), jnp.bfloat16)
    a_aug = a_aug.at[:, :_NUM_CONCEPTS].set((0.5 * A).astype(jnp.bfloat16))
    a_aug = a_aug.at[:, _NUM_CONCEPTS].set(jnp.bfloat16(1.0))
    d_half = (0.5 * D).astype(jnp.bfloat16)
    return pl.pallas_call(
        _fwd,
        grid=(nb,),
        in_specs=[
            pl.BlockSpec((_NUM_STUDENTS, 2 * _NUM_CONCEPTS), lambda q: (0, 0)),
            pl.BlockSpec((_QB, _NUM_CONCEPTS), lambda q: (q, 0)),
            pl.BlockSpec((1, 1, _QB), lambda q: (q, 0, 0)),
        ],
        out_specs=pl.BlockSpec((_NUM_STUDENTS, _QB), lambda q: (0, q)),
        out_shape=jax.ShapeDtypeStruct((_NUM_STUDENTS, _NUM_QUESTIONS),
                                       jnp.float32),
    )(a_aug, d_half, c3)
